# baseline (speedup 1.0000x reference)
import functools

import jax
import jax.numpy as jnp
from jax import lax
from jax.experimental import pallas as pl
from jax.experimental.pallas import tpu as pltpu
from jax.experimental.pallas import tpu_sc as plsc

_NC, _NS, _L = 2, 16, 16
_NW = _NC * _NS
_PCH = 16384


def kernel(x, table):
    batch, seq_len, dim = x.shape
    flat = batch * seq_len * dim
    tflat = seq_len * dim
    span = flat // _NW
    n_pieces = span // _PCH
    half = n_pieces // 2
    w_per_b = _NW // batch
    xf = x.reshape(flat)
    tf = table[:seq_len].reshape(tflat)

    mesh = plsc.VectorSubcoreMesh(core_axis_name="c", subcore_axis_name="s")

    vmem = lambda: pltpu.VMEM((_PCH,), jnp.float32)

    @functools.partial(
        pl.kernel,
        mesh=mesh,
        out_type=jax.ShapeDtypeStruct((flat,), jnp.float32),
        scratch_types=(
            (vmem(), vmem()),
            (vmem(), vmem()),
            (vmem(), vmem()),
            (pltpu.SemaphoreType.DMA,) * 2,
            (pltpu.SemaphoreType.DMA,) * 2,
        ),
    )
    def k(x_hbm, t_hbm, o_hbm, tbufs, xbufs, obufs, lsems, ssems):
        wid = lax.axis_index("s") * _NC + lax.axis_index("c")
        base = wid * span
        tbase = (wid % w_per_b) * span

        def issue_load(r, p):
            off = p * _PCH
            pltpu.async_copy(t_hbm.at[pl.ds(tbase + off, _PCH)], tbufs[r], lsems[r])
            pltpu.async_copy(x_hbm.at[pl.ds(base + off, _PCH)], xbufs[r], lsems[r])

        def wait_load(r):
            pltpu.make_async_copy(t_hbm.at[pl.ds(0, _PCH)], tbufs[r], lsems[r]).wait()
            pltpu.make_async_copy(x_hbm.at[pl.ds(0, _PCH)], xbufs[r], lsems[r]).wait()

        def issue_store(r, p):
            pltpu.async_copy(obufs[r], o_hbm.at[pl.ds(base + p * _PCH, _PCH)], ssems[r])

        def wait_store(r):
            pltpu.make_async_copy(obufs[r], o_hbm.at[pl.ds(0, _PCH)], ssems[r]).wait()

        def compute(r):
            @plsc.parallel_loop(0, _PCH, step=_L, unroll=8)
            def vec(v):
                sl = pl.ds(v, _L)
                obufs[r][sl] = xbufs[r][sl] + tbufs[r][sl]

        issue_load(0, 0)
        issue_load(1, 1)

        for r in range(2):
            wait_load(r)
            compute(r)
            issue_store(r, r)
            issue_load(r, r + 2)

        def body(g, c):
            for r in range(2):
                p = g * 2 + r
                wait_load(r)
                wait_store(r)
                compute(r)
                issue_store(r, p)
                issue_load(r, p + 2)
            return c

        lax.fori_loop(1, half - 1, body, 0)

        for r in range(2):
            p = (half - 1) * 2 + r
            wait_load(r)
            wait_store(r)
            compute(r)
            issue_store(r, p)
        for r in range(2):
            wait_store(r)

    out = k(xf, tf)
    return out.reshape(batch, seq_len, dim)


# speedup vs baseline: 4.5634x; 4.5634x over previous
import jax
import jax.numpy as jnp
from jax.experimental import pallas as pl
from jax.experimental.pallas import tpu as pltpu


_BS = 2048


def _add_kernel(x_ref, t_ref, o_ref):
    o_ref[...] = x_ref[...] + t_ref[...]


def kernel(x, table):
    batch, seq_len, dim = x.shape
    pos = table[:seq_len]
    grid = (seq_len // _BS, batch)
    return pl.pallas_call(
        _add_kernel,
        grid=grid,
        in_specs=[
            pl.BlockSpec((1, _BS, dim), lambda i, j: (j, i, 0)),
            pl.BlockSpec((_BS, dim), lambda i, j: (i, 0)),
        ],
        out_specs=pl.BlockSpec((1, _BS, dim), lambda i, j: (j, i, 0)),
        out_shape=jax.ShapeDtypeStruct((batch, seq_len, dim), x.dtype),
        compiler_params=pltpu.CompilerParams(
            dimension_semantics=("parallel", "arbitrary"),
        ),
    )(x, pos)
